# baseline (speedup 1.0000x reference)
import functools

import jax
import jax.numpy as jnp
from jax import lax
from jax.experimental import pallas as pl
from jax.experimental.pallas import tpu as pltpu
from jax.experimental.pallas import tpu_sc as plsc

BATCH = 16384
D = 64
H1 = 128
H2 = 64
NC = 2
NS = 16
NW = NC * NS
BPW = BATCH // NW


@functools.cache
def _make_gather_embeddings():
    mesh = plsc.VectorSubcoreMesh(core_axis_name="c", subcore_axis_name="s")

    @functools.partial(
        pl.kernel,
        out_type=(
            jax.ShapeDtypeStruct((BATCH, D), jnp.float32),
            jax.ShapeDtypeStruct((BATCH, D), jnp.float32),
        ),
        mesh=mesh,
        scratch_types=[
            pltpu.VMEM((BPW,), jnp.int32),
            pltpu.VMEM((BPW,), jnp.int32),
            pltpu.SemaphoreType.DMA,
            pltpu.SemaphoreType.DMA,
        ],
    )
    def gather_embeddings(uid_hbm, bid_hbm, utab_hbm, btab_hbm,
                          uout_hbm, bout_hbm,
                          uidx_v, bidx_v, usem, bsem):
        wid = lax.axis_index("s") * NC + lax.axis_index("c")
        base = wid * BPW

        def issue(idx_v, tab_hbm, out_hbm, sem):
            def grp(k, carry):
                v = idx_v[pl.ds(k * 16, 16)]
                for j in range(16):
                    pltpu.async_copy(
                        tab_hbm.at[pl.ds(v[j], 1)],
                        out_hbm.at[pl.ds(base + k * 16 + j, 1)],
                        sem,
                    )
                return carry

            lax.fori_loop(0, BPW // 16, grp, 0)

        pltpu.sync_copy(uid_hbm.at[pl.ds(base, BPW)], uidx_v)
        pltpu.sync_copy(bid_hbm.at[pl.ds(base, BPW)], bidx_v)
        issue(uidx_v, utab_hbm, uout_hbm, usem)
        issue(bidx_v, btab_hbm, bout_hbm, bsem)
        pltpu.make_async_copy(
            utab_hbm.at[pl.ds(0, BPW)], uout_hbm.at[pl.ds(base, BPW)], usem
        ).wait()
        pltpu.make_async_copy(
            btab_hbm.at[pl.ds(0, BPW)], bout_hbm.at[pl.ds(base, BPW)], bsem
        ).wait()

    return gather_embeddings


BT = 1024


def _mlp_body(u_ref, b_ref, w1u_ref, w1b_ref, b1_ref, w2_ref, b2_ref,
              w3_ref, b3_ref, out_ref):
    x1 = jnp.dot(u_ref[...], w1u_ref[...], preferred_element_type=jnp.float32)
    x1 += jnp.dot(b_ref[...], w1b_ref[...], preferred_element_type=jnp.float32)
    x1 = jnp.maximum(x1 + b1_ref[...], 0.0)
    x2 = jnp.dot(x1, w2_ref[...], preferred_element_type=jnp.float32)
    x2 = jnp.maximum(x2 + b2_ref[...], 0.0)
    out_ref[...] = jnp.sum(x2 * w3_ref[...], axis=1) + b3_ref[0, 0]


def kernel(user_id, book_id, user_table, book_table, W1, b1, W2, b2, W3, b3):
    uemb, bemb = _make_gather_embeddings()(user_id, book_id, user_table,
                                           book_table)
    w1t = W1.T
    out = pl.pallas_call(
        _mlp_body,
        grid=(BATCH // BT,),
        in_specs=[
            pl.BlockSpec((BT, D), lambda i: (i, 0)),
            pl.BlockSpec((BT, D), lambda i: (i, 0)),
            pl.BlockSpec((D, H1), lambda i: (0, 0)),
            pl.BlockSpec((D, H1), lambda i: (0, 0)),
            pl.BlockSpec((1, H1), lambda i: (0, 0)),
            pl.BlockSpec((H1, H2), lambda i: (0, 0)),
            pl.BlockSpec((1, H2), lambda i: (0, 0)),
            pl.BlockSpec((1, H2), lambda i: (0, 0)),
            pl.BlockSpec((1, 1), lambda i: (0, 0)),
        ],
        out_specs=pl.BlockSpec((BT,), lambda i: (i,)),
        out_shape=jax.ShapeDtypeStruct((BATCH,), jnp.float32),
    )(uemb, bemb, w1t[:D], w1t[D:], b1.reshape(1, H1), W2.T,
      b2.reshape(1, H2), W3, b3.reshape(1, 1))
    return out


# speedup vs baseline: 2.1511x; 2.1511x over previous
import functools

import jax
import jax.numpy as jnp
from jax import lax
from jax.experimental import pallas as pl
from jax.experimental.pallas import tpu as pltpu
from jax.experimental.pallas import tpu_sc as plsc

BATCH = 16384
D = 64
H1 = 128
H2 = 64
NC = 2
NS = 16
NW = NC * NS
BPW = BATCH // NW


@functools.cache
def _make_gather_embeddings():
    mesh = plsc.VectorSubcoreMesh(core_axis_name="c", subcore_axis_name="s")

    @functools.partial(
        pl.kernel,
        out_type=(
            jax.ShapeDtypeStruct((BATCH, D), jnp.float32),
            jax.ShapeDtypeStruct((BATCH, D), jnp.float32),
        ),
        mesh=mesh,
        scratch_types=[
            pltpu.VMEM((BPW,), jnp.int32),
            pltpu.VMEM((BPW, D), jnp.float32),
            pltpu.SemaphoreType.DMA,
        ],
    )
    def gather_embeddings(uid_hbm, bid_hbm, utab_hbm, btab_hbm,
                          uout_hbm, bout_hbm,
                          idx_v, rows_v, sem):
        wid = lax.axis_index("s") * NC + lax.axis_index("c")
        base = wid * BPW

        def one_table(idx_hbm, tab_hbm, out_hbm):
            pltpu.sync_copy(idx_hbm.at[pl.ds(base, BPW)], idx_v)

            def grp(k, carry):
                v = idx_v[pl.ds(k * 16, 16)]
                for j in range(16):
                    pltpu.async_copy(
                        tab_hbm.at[pl.ds(v[j], 1)],
                        rows_v.at[pl.ds(k * 16 + j, 1)],
                        sem,
                    )
                return carry

            lax.fori_loop(0, BPW // 16, grp, 0)
            pltpu.make_async_copy(
                tab_hbm.at[pl.ds(0, BPW)], rows_v, sem
            ).wait()
            pltpu.sync_copy(rows_v, out_hbm.at[pl.ds(base, BPW)])

        one_table(uid_hbm, utab_hbm, uout_hbm)
        one_table(bid_hbm, btab_hbm, bout_hbm)

    return gather_embeddings


BT = 1024


def _mlp_body(u_ref, b_ref, w1u_ref, w1b_ref, b1_ref, w2_ref, b2_ref,
              w3_ref, b3_ref, out_ref):
    x1 = jnp.dot(u_ref[...], w1u_ref[...], preferred_element_type=jnp.float32)
    x1 += jnp.dot(b_ref[...], w1b_ref[...], preferred_element_type=jnp.float32)
    x1 = jnp.maximum(x1 + b1_ref[...], 0.0)
    x2 = jnp.dot(x1, w2_ref[...], preferred_element_type=jnp.float32)
    x2 = jnp.maximum(x2 + b2_ref[...], 0.0)
    out_ref[...] = jnp.sum(x2 * w3_ref[...], axis=1) + b3_ref[0, 0]


def kernel(user_id, book_id, user_table, book_table, W1, b1, W2, b2, W3, b3):
    uemb, bemb = _make_gather_embeddings()(user_id, book_id, user_table,
                                           book_table)
    w1t = W1.T
    out = pl.pallas_call(
        _mlp_body,
        grid=(BATCH // BT,),
        in_specs=[
            pl.BlockSpec((BT, D), lambda i: (i, 0)),
            pl.BlockSpec((BT, D), lambda i: (i, 0)),
            pl.BlockSpec((D, H1), lambda i: (0, 0)),
            pl.BlockSpec((D, H1), lambda i: (0, 0)),
            pl.BlockSpec((1, H1), lambda i: (0, 0)),
            pl.BlockSpec((H1, H2), lambda i: (0, 0)),
            pl.BlockSpec((1, H2), lambda i: (0, 0)),
            pl.BlockSpec((1, H2), lambda i: (0, 0)),
            pl.BlockSpec((1, 1), lambda i: (0, 0)),
        ],
        out_specs=pl.BlockSpec((BT,), lambda i: (i,)),
        out_shape=jax.ShapeDtypeStruct((BATCH,), jnp.float32),
    )(uemb, bemb, w1t[:D], w1t[D:], b1.reshape(1, H1), W2.T,
      b2.reshape(1, H2), W3, b3.reshape(1, 1))
    return out


# speedup vs baseline: 2.1687x; 1.0082x over previous
import functools

import jax
import jax.numpy as jnp
from jax import lax
from jax.experimental import pallas as pl
from jax.experimental.pallas import tpu as pltpu
from jax.experimental.pallas import tpu_sc as plsc

BATCH = 16384
D = 64
H1 = 128
H2 = 64
NC = 2
NS = 16
NW = NC * NS
BPW = BATCH // NW
NSEM = 4


@functools.cache
def _make_gather_embeddings():
    mesh = plsc.VectorSubcoreMesh(core_axis_name="c", subcore_axis_name="s")

    @functools.partial(
        pl.kernel,
        out_type=(
            jax.ShapeDtypeStruct((BATCH, D), jnp.float32),
            jax.ShapeDtypeStruct((BATCH, D), jnp.float32),
        ),
        mesh=mesh,
        scratch_types=[
            pltpu.VMEM((BPW,), jnp.int32),
            pltpu.VMEM((BPW, D), jnp.float32),
            pltpu.SemaphoreType.DMA,
            pltpu.SemaphoreType.DMA,
            pltpu.SemaphoreType.DMA,
            pltpu.SemaphoreType.DMA,
        ],
    )
    def gather_embeddings(uid_hbm, bid_hbm, utab_hbm, btab_hbm,
                          uout_hbm, bout_hbm,
                          idx_v, rows_v, sem0, sem1, sem2, sem3):
        wid = lax.axis_index("s") * NC + lax.axis_index("c")
        base = wid * BPW
        sems = (sem0, sem1, sem2, sem3)

        def one_table(idx_hbm, tab_hbm, out_hbm):
            pltpu.sync_copy(idx_hbm.at[pl.ds(base, BPW)], idx_v)

            def grp(k, carry):
                v = idx_v[pl.ds(k * 16, 16)]
                for j in range(16):
                    pltpu.async_copy(
                        tab_hbm.at[pl.ds(v[j], 1)],
                        rows_v.at[pl.ds(k * 16 + j, 1)],
                        sems[j % NSEM],
                    )
                return carry

            lax.fori_loop(0, BPW // 16, grp, 0)
            for s in range(NSEM):
                pltpu.make_async_copy(
                    tab_hbm.at[pl.ds(0, BPW // NSEM)],
                    rows_v.at[pl.ds(0, BPW // NSEM)],
                    sems[s],
                ).wait()
            pltpu.sync_copy(rows_v, out_hbm.at[pl.ds(base, BPW)])

        one_table(uid_hbm, utab_hbm, uout_hbm)
        one_table(bid_hbm, btab_hbm, bout_hbm)

    return gather_embeddings


BT = 1024


def _mlp_body(u_ref, b_ref, w1u_ref, w1b_ref, b1_ref, w2_ref, b2_ref,
              w3_ref, b3_ref, out_ref):
    x1 = jnp.dot(u_ref[...], w1u_ref[...], preferred_element_type=jnp.float32)
    x1 += jnp.dot(b_ref[...], w1b_ref[...], preferred_element_type=jnp.float32)
    x1 = jnp.maximum(x1 + b1_ref[...], 0.0)
    x2 = jnp.dot(x1, w2_ref[...], preferred_element_type=jnp.float32)
    x2 = jnp.maximum(x2 + b2_ref[...], 0.0)
    out_ref[...] = jnp.sum(x2 * w3_ref[...], axis=1) + b3_ref[0, 0]


def kernel(user_id, book_id, user_table, book_table, W1, b1, W2, b2, W3, b3):
    uemb, bemb = _make_gather_embeddings()(user_id, book_id, user_table,
                                           book_table)
    w1t = W1.T
    out = pl.pallas_call(
        _mlp_body,
        grid=(BATCH // BT,),
        in_specs=[
            pl.BlockSpec((BT, D), lambda i: (i, 0)),
            pl.BlockSpec((BT, D), lambda i: (i, 0)),
            pl.BlockSpec((D, H1), lambda i: (0, 0)),
            pl.BlockSpec((D, H1), lambda i: (0, 0)),
            pl.BlockSpec((1, H1), lambda i: (0, 0)),
            pl.BlockSpec((H1, H2), lambda i: (0, 0)),
            pl.BlockSpec((1, H2), lambda i: (0, 0)),
            pl.BlockSpec((1, H2), lambda i: (0, 0)),
            pl.BlockSpec((1, 1), lambda i: (0, 0)),
        ],
        out_specs=pl.BlockSpec((BT,), lambda i: (i,)),
        out_shape=jax.ShapeDtypeStruct((BATCH,), jnp.float32),
    )(uemb, bemb, w1t[:D], w1t[D:], b1.reshape(1, H1), W2.T,
      b2.reshape(1, H2), W3, b3.reshape(1, 1))
    return out
